# Initial kernel scaffold; baseline (speedup 1.0000x reference)
import functools

import jax
import jax.numpy as jnp
import numpy as np
from jax import lax
from jax.experimental import pallas as pl
from jax.experimental.pallas import tpu as pltpu
from jax.experimental.pallas import tpu_sc as plsc

_VOCAB = 100000
_D = 64
_B = 16384
_K = 20
_P = _K + 1
_NW = 32
_BW = _B // _NW
_PW = _BW * _P
_CB = 16
_CP = _CB * _P
_NCHUNK = _BW // _CB
_GSZ = 112
_NG = _CP // _GSZ


def _sc_scores(tidx, cnidx, temb, cemb):
  mesh = plsc.VectorSubcoreMesh(core_axis_name="c", subcore_axis_name="s")

  @functools.partial(
      pl.kernel,
      out_type=jax.ShapeDtypeStruct((_B * _P,), jnp.float32),
      mesh=mesh,
      scratch_types=[
          pltpu.VMEM((_BW,), jnp.int32),
          pltpu.VMEM((_PW,), jnp.int32),
          pltpu.VMEM((_BW, _D), jnp.float32),
          pltpu.VMEM((_CP, _D), jnp.float32),
          pltpu.VMEM((_PW,), jnp.float32),
          pltpu.SemaphoreType.DMA,
          pltpu.SemaphoreType.DMA,
      ],
  )
  def body(tidx_hbm, cnidx_hbm, temb_hbm, cemb_hbm, out_hbm,
           tidx_v, cnidx_v, trows_v, cbuf_v, scores_v, sem_t, sem_c):
    wid = lax.axis_index("s") * 2 + lax.axis_index("c")
    b0 = wid * _BW
    pltpu.sync_copy(tidx_hbm.at[pl.ds(b0, _BW)], tidx_v)
    pltpu.sync_copy(cnidx_hbm.at[pl.ds(b0 * _P, _PW)], cnidx_v)
    tcopies = [
        pltpu.async_copy(
            temb_hbm.at[tidx_v.at[pl.ds(i * 128, 128)]],
            trows_v.at[pl.ds(i * 128, 128)], sem_t)
        for i in range(_BW // 128)
    ]
    for cp in tcopies:
      cp.wait()

    def chunk_body(c, carry):
      ccopies = [
          pltpu.async_copy(
              cemb_hbm.at[cnidx_v.at[pl.ds(c * _CP + g * _GSZ, _GSZ)]],
              cbuf_v.at[pl.ds(g * _GSZ, _GSZ)], sem_c)
          for g in range(_NG)
      ]
      for cp in ccopies:
        cp.wait()

      def b_body(bl, carry2):
        b = c * _CB + bl
        t = [trows_v[b, pl.ds(16 * q, 16)] for q in range(4)]
        for j in range(_P):
          p = bl * _P + j
          acc = t[0] * cbuf_v[p, pl.ds(0, 16)]
          for q in range(1, 4):
            acc = acc + t[q] * cbuf_v[p, pl.ds(16 * q, 16)]
          s = jnp.sum(acc)
          scores_v[c * _CP + p] = s if j == 0 else -s
        return carry2

      lax.fori_loop(0, _CB, b_body, 0)
      return carry

    lax.fori_loop(0, _NCHUNK, chunk_body, 0)
    pltpu.sync_copy(scores_v, out_hbm.at[pl.ds(wid * _PW, _PW)])

  return body(tidx, cnidx, temb, cemb)


_RCOLS = 256
_RROWS = _B * _P // _RCOLS


def _tc_loss(scores):
  def body(x_ref, o_ref):
    x = x_ref[...]
    o_ref[0, 0] = -jnp.sum(jax.nn.log_sigmoid(x)) / np.float32(_B)

  return pl.pallas_call(
      body,
      out_shape=jax.ShapeDtypeStruct((1, 1), jnp.float32),
      out_specs=pl.BlockSpec(memory_space=pltpu.SMEM),
  )(scores.reshape(_RROWS, _RCOLS))


def kernel(target, context, negatives, target_emb, context_emb):
  tidx = target.astype(jnp.int32)
  cnidx = jnp.concatenate(
      [context.astype(jnp.int32)[:, None], negatives.astype(jnp.int32)],
      axis=1).reshape(-1)
  scores = _sc_scores(tidx, cnidx, target_emb, context_emb)
  return _tc_loss(scores)[0, 0]


# speedup vs baseline: 3.1392x; 3.1392x over previous
import functools

import jax
import jax.numpy as jnp
import numpy as np
from jax import lax
from jax.experimental import pallas as pl
from jax.experimental.pallas import tpu as pltpu
from jax.experimental.pallas import tpu_sc as plsc

_VOCAB = 100000
_D = 64
_B = 16384
_K = 20
_P = _K + 1
_NW = 32
_BW = _B // _NW
_PW = _BW * _P
_CB = 16
_CP = _CB * _P
_NCHUNK = _BW // _CB
_GSZ = 112
_NG = _CP // _GSZ


def _sc_scores(tidx, cnidx, temb, cemb):
  mesh = plsc.VectorSubcoreMesh(core_axis_name="c", subcore_axis_name="s")

  @functools.partial(
      pl.kernel,
      out_type=jax.ShapeDtypeStruct((_B * _P,), jnp.float32),
      mesh=mesh,
      compiler_params=pltpu.CompilerParams(
          needs_layout_passes=False, use_tc_tiling_on_sc=False),
      scratch_types=[
          pltpu.VMEM((_BW,), jnp.int32),
          pltpu.VMEM((_PW,), jnp.int32),
          pltpu.VMEM((_BW, _D), jnp.float32),
          pltpu.VMEM((_CP, _D), jnp.float32),
          pltpu.VMEM((_PW,), jnp.float32),
          pltpu.SemaphoreType.DMA,
          pltpu.SemaphoreType.DMA,
      ],
  )
  def body(tidx_hbm, cnidx_hbm, temb_hbm, cemb_hbm, out_hbm,
           tidx_v, cnidx_v, trows_v, cbuf_v, scores_v, sem_t, sem_c):
    wid = lax.axis_index("s") * 2 + lax.axis_index("c")
    b0 = wid * _BW
    pltpu.sync_copy(tidx_hbm.at[pl.ds(b0, _BW)], tidx_v)
    pltpu.sync_copy(cnidx_hbm.at[pl.ds(b0 * _P, _PW)], cnidx_v)
    tcopies = [
        pltpu.async_copy(
            temb_hbm.at[tidx_v.at[pl.ds(i * 128, 128)]],
            trows_v.at[pl.ds(i * 128, 128)], sem_t)
        for i in range(_BW // 128)
    ]
    for cp in tcopies:
      cp.wait()

    def chunk_body(c, carry):
      ccopies = [
          pltpu.async_copy(
              cemb_hbm.at[cnidx_v.at[pl.ds(c * _CP + g * _GSZ, _GSZ)]],
              cbuf_v.at[pl.ds(g * _GSZ, _GSZ)], sem_c)
          for g in range(_NG)
      ]
      for cp in ccopies:
        cp.wait()

      lane0 = lax.iota(jnp.int32, 16) == 0

      def b_body(bl, carry2):
        b = c * _CB + bl
        t = [trows_v[b, pl.ds(16 * q, 16)] for q in range(4)]
        for j in range(_P):
          p = bl * _P + j
          acc = t[0] * cbuf_v[p, pl.ds(0, 16)]
          for q in range(1, 4):
            acc = acc + t[q] * cbuf_v[p, pl.ds(16 * q, 16)]
          s = jnp.sum(acc)
          sv = jnp.full((16,), s if j == 0 else -s, jnp.float32)
          addr = jnp.full((16,), c * _CP + p, jnp.int32)
          plsc.store_scatter(scores_v, [addr], sv, mask=lane0)
        return carry2

      lax.fori_loop(0, _CB, b_body, 0)
      return carry

    lax.fori_loop(0, _NCHUNK, chunk_body, 0)
    pltpu.sync_copy(scores_v, out_hbm.at[pl.ds(wid * _PW, _PW)])

  return body(tidx, cnidx, temb, cemb)


_RCOLS = 256
_RROWS = _B * _P // _RCOLS


def _tc_loss(scores):
  def body(x_ref, o_ref):
    x = x_ref[...]
    o_ref[0, 0] = -jnp.sum(jax.nn.log_sigmoid(x)) / np.float32(_B)

  return pl.pallas_call(
      body,
      out_shape=jax.ShapeDtypeStruct((1, 1), jnp.float32),
      out_specs=pl.BlockSpec(memory_space=pltpu.SMEM),
  )(scores.reshape(_RROWS, _RCOLS))


def kernel(target, context, negatives, target_emb, context_emb):
  tidx = target.astype(jnp.int32)
  cnidx = jnp.concatenate(
      [context.astype(jnp.int32)[:, None], negatives.astype(jnp.int32)],
      axis=1).reshape(-1)
  scores = _sc_scores(tidx, cnidx, target_emb, context_emb)
  return _tc_loss(scores)[0, 0]


# speedup vs baseline: 4.7872x; 1.5250x over previous
import functools

import jax
import jax.numpy as jnp
import numpy as np
from jax import lax
from jax.experimental import pallas as pl
from jax.experimental.pallas import tpu as pltpu
from jax.experimental.pallas import tpu_sc as plsc

_VOCAB = 100000
_D = 64
_B = 16384
_K = 20
_P = _K + 1
_NW = 32
_BW = _B // _NW
_PW = _BW * _P
_CB = 16
_CP = _CB * _P
_NCHUNK = _BW // _CB
_GSZ = 112
_NG = _CP // _GSZ


def _sc_scores(tidx, cnidx, temb, cemb):
  mesh = plsc.VectorSubcoreMesh(core_axis_name="c", subcore_axis_name="s")

  @functools.partial(
      pl.kernel,
      out_type=jax.ShapeDtypeStruct((_B * _P,), jnp.float32),
      mesh=mesh,
      compiler_params=pltpu.CompilerParams(
          needs_layout_passes=False, use_tc_tiling_on_sc=False),
      scratch_types=[
          pltpu.VMEM((_BW,), jnp.int32),
          pltpu.VMEM((_PW,), jnp.int32),
          pltpu.VMEM((_BW, _D), jnp.float32),
          pltpu.VMEM((2, _CP, _D), jnp.float32),
          pltpu.VMEM((_PW,), jnp.float32),
          pltpu.SemaphoreType.DMA,
          pltpu.SemaphoreType.DMA((2,)),
      ],
  )
  def body(tidx_hbm, cnidx_hbm, temb_hbm, cemb_hbm, out_hbm,
           tidx_v, cnidx_v, trows_v, cbuf_v, scores_v, sem_t, sem_c):
    wid = lax.axis_index("s") * 2 + lax.axis_index("c")
    b0 = wid * _BW
    pltpu.sync_copy(tidx_hbm.at[pl.ds(b0, _BW)], tidx_v)
    pltpu.sync_copy(cnidx_hbm.at[pl.ds(b0 * _P, _PW)], cnidx_v)

    def chunk_copies(c, buf, start):
      for g in range(_NG):
        desc = pltpu.make_async_copy(
            cemb_hbm.at[cnidx_v.at[pl.ds(c * _CP + g * _GSZ, _GSZ)]],
            cbuf_v.at[buf, pl.ds(g * _GSZ, _GSZ)], sem_c.at[buf])
        if start:
          desc.start()
        else:
          desc.wait()

    tcopies = [
        pltpu.async_copy(
            temb_hbm.at[tidx_v.at[pl.ds(i * 128, 128)]],
            trows_v.at[pl.ds(i * 128, 128)], sem_t)
        for i in range(_BW // 128)
    ]
    chunk_copies(0, 0, True)
    for cp in tcopies:
      cp.wait()

    lane15 = lax.iota(jnp.int32, 16) == 15

    def chunk_body(c, carry):
      buf = c & 1

      @pl.when(c + 1 < _NCHUNK)
      def _prefetch():
        chunk_copies(c + 1, 1 - buf, True)

      chunk_copies(c, buf, False)

      @plsc.parallel_loop(0, _CB, unroll=2)
      def b_body(bl):
        b = c * _CB + bl
        t = [trows_v[b, pl.ds(16 * q, 16)] for q in range(4)]
        tn = [-x for x in t]
        pbase = c * _CP + bl * _P
        for j in range(_P):
          tt = t if j == 0 else tn
          p = bl * _P + j
          acc = tt[0] * cbuf_v[buf, p, pl.ds(0, 16)]
          for q in range(1, 4):
            acc = acc + tt[q] * cbuf_v[buf, p, pl.ds(16 * q, 16)]
          sv = plsc.cumsum(acc)
          addr = jnp.full((16,), pbase + j, jnp.int32)
          plsc.store_scatter(scores_v, [addr], sv, mask=lane15)

      return carry

    lax.fori_loop(0, _NCHUNK, chunk_body, 0)
    pltpu.sync_copy(scores_v, out_hbm.at[pl.ds(wid * _PW, _PW)])

  return body(tidx, cnidx, temb, cemb)


_RCOLS = 256
_RROWS = _B * _P // _RCOLS


def _tc_loss(scores):
  def body(x_ref, o_ref):
    x = x_ref[...]
    o_ref[0, 0] = -jnp.sum(jax.nn.log_sigmoid(x)) / np.float32(_B)

  return pl.pallas_call(
      body,
      out_shape=jax.ShapeDtypeStruct((1, 1), jnp.float32),
      out_specs=pl.BlockSpec(memory_space=pltpu.SMEM),
  )(scores.reshape(_RROWS, _RCOLS))


def kernel(target, context, negatives, target_emb, context_emb):
  tidx = target.astype(jnp.int32)
  cnidx = jnp.concatenate(
      [context.astype(jnp.int32)[:, None], negatives.astype(jnp.int32)],
      axis=1).reshape(-1)
  scores = _sc_scores(tidx, cnidx, target_emb, context_emb)
  return _tc_loss(scores)[0, 0]
